# baseline (speedup 1.0000x reference)
import math

import jax
import jax.numpy as jnp
from jax.experimental import pallas as pl

_D = 128
_NH = 4
_DH = _D // _NH
_R = 50
_RP = 56
_MEM = 100000
_SCALE = math.sqrt(_D // _NH)

_BLK = 64


def _attn_body(q_ref, kin_ref, vin_ref, idx_ref, wqt_ref, bq_ref, wkt_ref,
               bk_ref, wvt_ref, bv_ref, dh_ref, dl_ref, h_ref, l_ref):
    q = q_ref[...]
    kin = kin_ref[...]
    vin = vin_ref[...]
    idx = idx_ref[...]
    mask = (idx != 0)
    maskf = mask.astype(jnp.float32)

    qp = jnp.dot(q, wqt_ref[...], preferred_element_type=jnp.float32) + bq_ref[...]
    kp = (jnp.dot(kin.reshape(_BLK * _RP, _D), wkt_ref[...],
                  preferred_element_type=jnp.float32)
          + bk_ref[...]).reshape(_BLK, _RP, _D)
    vp = (jnp.dot(vin.reshape(_BLK * _RP, _D), wvt_ref[...],
                  preferred_element_type=jnp.float32)
          + bv_ref[...]).reshape(_BLK, _RP, _D)

    tmp = kp * qp[:, None, :]

    pfull_parts = []
    for h in range(_NH):
        lg = jnp.sum(tmp[:, :, h * _DH:(h + 1) * _DH], axis=2) / _SCALE
        lg = jnp.where(mask, lg, -1e30)
        m = jnp.max(lg, axis=1, keepdims=True)
        e = jnp.exp(lg - m) * maskf
        s = jnp.sum(e, axis=1, keepdims=True)
        p = e / jnp.maximum(s, 1e-30)
        pfull_parts.append(jnp.broadcast_to(p[:, :, None], (_BLK, _RP, _DH)))
    pfull = jnp.concatenate(pfull_parts, axis=2)

    h_out = jnp.sum(pfull * kp, axis=1)
    l_out = jnp.sum(pfull * vp, axis=1)

    empty = (jnp.sum(maskf, axis=1, keepdims=True) == 0.0)
    h_ref[...] = jnp.where(empty, dh_ref[...], h_out)
    l_ref[...] = jnp.where(empty, dl_ref[...], l_out)


def _attention(q, k_in, v_in, idx, WqT, bq, WkT, bk, WvT, bv, dh, dl):
    num = q.shape[0]
    grid = num // _BLK
    full = lambda i: (0, 0)
    h, l = pl.pallas_call(
        _attn_body,
        grid=(grid,),
        in_specs=[
            pl.BlockSpec((_BLK, _D), lambda i: (i, 0)),
            pl.BlockSpec((_BLK, _RP, _D), lambda i: (i, 0, 0)),
            pl.BlockSpec((_BLK, _RP, _D), lambda i: (i, 0, 0)),
            pl.BlockSpec((_BLK, _RP), lambda i: (i, 0)),
            pl.BlockSpec((_D, _D), full),
            pl.BlockSpec((1, _D), full),
            pl.BlockSpec((_D, _D), full),
            pl.BlockSpec((1, _D), full),
            pl.BlockSpec((_D, _D), full),
            pl.BlockSpec((1, _D), full),
            pl.BlockSpec((1, _D), full),
            pl.BlockSpec((1, _D), full),
        ],
        out_specs=[
            pl.BlockSpec((_BLK, _D), lambda i: (i, 0)),
            pl.BlockSpec((_BLK, _D), lambda i: (i, 0)),
        ],
        out_shape=[
            jax.ShapeDtypeStruct((num, _D), jnp.float32),
            jax.ShapeDtypeStruct((num, _D), jnp.float32),
        ],
    )(q, k_in, v_in, idx, WqT, bq, WkT, bk, WvT, bv, dh, dl)
    return h, l


def kernel(query_h, doc_idx, word_idx, word_mat, put_word_idx, h_put, l_put,
           default_h, default_l, Wq, bq, Wk, bk, Wv, bv):
    b, s, d = query_h.shape
    num = b * s
    inv = jnp.zeros((_MEM,), jnp.int32).at[put_word_idx].set(
        jnp.arange(_MEM, dtype=jnp.int32))
    idx = word_mat[doc_idx[0]][word_idx.reshape(-1)].astype(jnp.int32)
    idxp = jnp.pad(idx, ((0, 0), (0, _RP - _R)))
    fidx = inv[idxp]
    k_in = h_put[fidx]
    v_in = l_put[fidx]
    q = query_h.reshape(num, d)
    h, l = _attention(q, k_in, v_in, idxp,
                      Wq.T, bq.reshape(1, d), Wk.T, bk.reshape(1, d),
                      Wv.T, bv.reshape(1, d),
                      default_h.reshape(1, d), default_l.reshape(1, d))
    return h.reshape(b, s, d), l.reshape(b, s, d)


# speedup vs baseline: 1.7089x; 1.7089x over previous
import functools
import math

import jax
import jax.numpy as jnp
from jax import lax
from jax.experimental import pallas as pl
from jax.experimental.pallas import tpu as pltpu
from jax.experimental.pallas import tpu_sc as plsc

_D = 128
_NH = 4
_DH = _D // _NH
_R = 50
_RP = 56
_MEM = 100000
_NUM = 2048
_SCALE = math.sqrt(_D // _NH)

_NW = 32
_QW = _NUM // _NW
_PCH = 128
_NFULL = _MEM // _PCH
_PTAIL = _MEM - _NFULL * _PCH

_BLK = 64


def _mesh():
    return plsc.VectorSubcoreMesh(core_axis_name="c", subcore_axis_name="s")


def _wid():
    return lax.axis_index("s") * 2 + lax.axis_index("c")


@functools.partial(
    pl.kernel, mesh=_mesh(),
    out_type=[
        jax.ShapeDtypeStruct((_MEM, _D), jnp.float32),
        jax.ShapeDtypeStruct((_MEM, _D), jnp.float32),
    ],
    scratch_types=[
        pltpu.VMEM((2, _PCH), jnp.int32),
        pltpu.VMEM((2, _PCH, _D), jnp.float32),
        pltpu.VMEM((2, _PCH, _D), jnp.float32),
        pltpu.SemaphoreType.DMA,
        pltpu.SemaphoreType.DMA,
    ],
)
def _sc_put(put_hbm, hput_hbm, lput_hbm, memh_hbm, meml_hbm,
            pbuf, hbuf, lbuf, s_h, s_l):
    wid = _wid()
    nloop = (_NFULL + _NW - 1) // _NW
    for j in range(nloop):
        @pl.when(wid + _NW * j < _NFULL)
        def _():
            b = j % 2
            base = (wid + _NW * j) * _PCH
            pltpu.sync_copy(put_hbm.at[pl.ds(base, _PCH)], pbuf.at[b])
            pltpu.sync_copy(hput_hbm.at[pl.ds(base, _PCH)], hbuf.at[b])
            pltpu.sync_copy(lput_hbm.at[pl.ds(base, _PCH)], lbuf.at[b])
            idx = pbuf.at[b]
            pltpu.async_copy(hbuf.at[b], memh_hbm.at[idx], s_h).wait()
            pltpu.async_copy(lbuf.at[b], meml_hbm.at[idx], s_l).wait()

    @pl.when(wid == _NW - 1)
    def _():
        base = _NFULL * _PCH
        pltpu.sync_copy(put_hbm.at[pl.ds(base, _PTAIL)],
                        pbuf.at[0, pl.ds(0, _PTAIL)])
        pltpu.sync_copy(hput_hbm.at[pl.ds(base, _PTAIL)],
                        hbuf.at[0, pl.ds(0, _PTAIL)])
        pltpu.sync_copy(lput_hbm.at[pl.ds(base, _PTAIL)],
                        lbuf.at[0, pl.ds(0, _PTAIL)])
        idx = pbuf.at[0, pl.ds(0, _PTAIL)]
        pltpu.async_copy(hbuf.at[0, pl.ds(0, _PTAIL)],
                         memh_hbm.at[idx], s_h).wait()
        pltpu.async_copy(lbuf.at[0, pl.ds(0, _PTAIL)],
                         meml_hbm.at[idx], s_l).wait()


@functools.partial(
    pl.kernel, mesh=_mesh(),
    out_type=[
        jax.ShapeDtypeStruct((_NW, _QW, _D), jnp.int32),
        jax.ShapeDtypeStruct((_NUM * _RP, _D), jnp.float32),
        jax.ShapeDtypeStruct((_NUM * _RP, _D), jnp.float32),
    ],
    scratch_types=[
        pltpu.VMEM((_QW,), jnp.int32),
        pltpu.VMEM((_QW, _D), jnp.int32),
        pltpu.VMEM((2, _RP, _D), jnp.float32),
        pltpu.VMEM((2, _RP, _D), jnp.float32),
        pltpu.SemaphoreType.DMA,
        pltpu.SemaphoreType.DMA,
        pltpu.SemaphoreType.DMA,
        pltpu.SemaphoreType.DMA,
    ],
)
def _sc_get(wi_hbm, wm2p_hbm, memh_hbm, meml_hbm,
            idxout_hbm, kin_hbm, vin_hbm,
            wibuf, idx50, hring, lring, s_i, s_g, s_h, s_l):
    wid = _wid()
    pltpu.sync_copy(wi_hbm.at[pl.ds(wid * _QW, _QW)], wibuf)
    pltpu.async_copy(wm2p_hbm.at[wibuf], idx50, s_g).wait()
    out_h = pltpu.async_copy(idx50, idxout_hbm.at[wid], s_i)

    pend = [None, None]
    for i in range(_QW):
        b = i % 2
        if pend[b] is not None:
            for h in pend[b]:
                h.wait()
        slots = idx50.at[i, pl.ds(0, _RP)]
        pltpu.async_copy(memh_hbm.at[slots], hring.at[b], s_g).wait()
        pltpu.async_copy(meml_hbm.at[slots], lring.at[b], s_g).wait()
        dst = pl.ds((wid * _QW + i) * _RP, _RP)
        pend[b] = [
            pltpu.async_copy(hring.at[b], kin_hbm.at[dst], s_h),
            pltpu.async_copy(lring.at[b], vin_hbm.at[dst], s_l),
        ]
    for p in pend:
        if p is not None:
            for h in p:
                h.wait()
    out_h.wait()


def _attn_body(q_ref, kin_ref, vin_ref, idx_ref, wqt_ref, bq_ref, wkt_ref,
               bk_ref, wvt_ref, bv_ref, dh_ref, dl_ref, h_ref, l_ref):
    q = q_ref[...]
    kin = kin_ref[...]
    vin = vin_ref[...]
    idx = idx_ref[...][:, :_RP]
    mask = (idx != 0)
    maskf = mask.astype(jnp.float32)

    qp = jnp.dot(q, wqt_ref[...], preferred_element_type=jnp.float32) + bq_ref[...]
    kp = (jnp.dot(kin.reshape(_BLK * _RP, _D), wkt_ref[...],
                  preferred_element_type=jnp.float32)
          + bk_ref[...]).reshape(_BLK, _RP, _D)
    vp = (jnp.dot(vin.reshape(_BLK * _RP, _D), wvt_ref[...],
                  preferred_element_type=jnp.float32)
          + bv_ref[...]).reshape(_BLK, _RP, _D)

    tmp = kp * qp[:, None, :]

    pfull_parts = []
    for h in range(_NH):
        lg = jnp.sum(tmp[:, :, h * _DH:(h + 1) * _DH], axis=2) / _SCALE
        lg = jnp.where(mask, lg, -1e30)
        m = jnp.max(lg, axis=1, keepdims=True)
        e = jnp.exp(lg - m) * maskf
        s = jnp.sum(e, axis=1, keepdims=True)
        p = e / jnp.maximum(s, 1e-30)
        pfull_parts.append(jnp.broadcast_to(p[:, :, None], (_BLK, _RP, _DH)))
    pfull = jnp.concatenate(pfull_parts, axis=2)

    h_out = jnp.sum(pfull * kp, axis=1)
    l_out = jnp.sum(pfull * vp, axis=1)

    empty = (jnp.sum(maskf, axis=1, keepdims=True) == 0.0)
    h_ref[...] = jnp.where(empty, dh_ref[...], h_out)
    l_ref[...] = jnp.where(empty, dl_ref[...], l_out)


def _attention(q, k_in, v_in, idx, WqT, bq, WkT, bk, WvT, bv, dh, dl):
    num = q.shape[0]
    grid = num // _BLK
    full = lambda i: (0, 0)
    h, l = pl.pallas_call(
        _attn_body,
        grid=(grid,),
        in_specs=[
            pl.BlockSpec((_BLK, _D), lambda i: (i, 0)),
            pl.BlockSpec((_BLK, _RP, _D), lambda i: (i, 0, 0)),
            pl.BlockSpec((_BLK, _RP, _D), lambda i: (i, 0, 0)),
            pl.BlockSpec((_BLK, _D), lambda i: (i, 0)),
            pl.BlockSpec((_D, _D), full),
            pl.BlockSpec((1, _D), full),
            pl.BlockSpec((_D, _D), full),
            pl.BlockSpec((1, _D), full),
            pl.BlockSpec((_D, _D), full),
            pl.BlockSpec((1, _D), full),
            pl.BlockSpec((1, _D), full),
            pl.BlockSpec((1, _D), full),
        ],
        out_specs=[
            pl.BlockSpec((_BLK, _D), lambda i: (i, 0)),
            pl.BlockSpec((_BLK, _D), lambda i: (i, 0)),
        ],
        out_shape=[
            jax.ShapeDtypeStruct((num, _D), jnp.float32),
            jax.ShapeDtypeStruct((num, _D), jnp.float32),
        ],
    )(q, k_in, v_in, idx, WqT, bq, WkT, bk, WvT, bv, dh, dl)
    return h, l


def kernel(query_h, doc_idx, word_idx, word_mat, put_word_idx, h_put, l_put,
           default_h, default_l, Wq, bq, Wk, bk, Wv, bv):
    b, s, d = query_h.shape
    num = b * s
    mem_h, mem_l = _sc_put(put_word_idx.astype(jnp.int32), h_put, l_put)

    wm2p = jnp.pad(word_mat.reshape(-1, _R).astype(jnp.int32),
                   ((0, 0), (0, _D - _R)))
    wi = (word_idx.reshape(-1) + doc_idx[0] * 20000).astype(jnp.int32)
    idxg, k_in, v_in = _sc_get(wi, wm2p, mem_h, mem_l)

    q = query_h.reshape(num, d)
    h, l = _attention(q, k_in.reshape(num, _RP, d), v_in.reshape(num, _RP, d),
                      idxg.reshape(num, d),
                      Wq.T, bq.reshape(1, d), Wk.T, bk.reshape(1, d),
                      Wv.T, bv.reshape(1, d),
                      default_h.reshape(1, d), default_l.reshape(1, d))
    return h.reshape(b, s, d), l.reshape(b, s, d)
